# baseline (speedup 1.0000x reference)
import functools

import jax
import jax.numpy as jnp
from jax import lax
from jax.experimental import pallas as pl
from jax.experimental.pallas import tpu as pltpu
from jax.experimental.pallas import tpu_sc as plsc

_N_ROWS, _N_COLS = 8192, 4096
_LANES = 16
_NUM_WORKERS = 32
_GROUPS = _N_ROWS // _NUM_WORKERS // _LANES
_CC = 2048


@functools.cache
def _sc_cumsum_call():
    mesh = plsc.VectorSubcoreMesh(core_axis_name="c", subcore_axis_name="s")

    @functools.partial(
        pl.kernel,
        mesh=mesh,
        compiler_params=pltpu.CompilerParams(
            needs_layout_passes=False, use_tc_tiling_on_sc=False),
        out_type=jax.ShapeDtypeStruct((_N_ROWS, _N_COLS), jnp.float32),
        scratch_types=[pltpu.VMEM((_LANES, _CC), jnp.float32),
                       pltpu.VMEM((_LANES, _CC), jnp.float32)],
    )
    def sc_cumsum(x_hbm, o_hbm, ibuf, obuf):
        wid = lax.axis_index("s") * 2 + lax.axis_index("c")
        lanes = lax.iota(jnp.int32, 16)

        def group_body(g, _):
            row0 = (wid * _GROUPS + g) * _LANES

            def chunk_body(c, carry):
                col0 = c * _CC
                pltpu.sync_copy(
                    x_hbm.at[pl.ds(row0, _LANES), pl.ds(col0, _CC)], ibuf)

                def col_body(j, acc):
                    jv = jnp.zeros((_LANES,), jnp.int32) + j
                    acc = acc + plsc.load_gather(ibuf, [lanes, jv])
                    plsc.store_scatter(obuf, [lanes, jv], acc)
                    return acc

                carry = plsc.parallel_loop(
                    0, _CC, unroll=8, carry=carry)(col_body)

                pltpu.sync_copy(
                    obuf, o_hbm.at[pl.ds(row0, _LANES), pl.ds(col0, _CC)])
                return carry

            lax.fori_loop(0, _N_COLS // _CC, chunk_body,
                          jnp.zeros((_LANES,), jnp.float32))
            return 0

        lax.fori_loop(0, _GROUPS, group_body, 0)

    return sc_cumsum


def kernel(x):
    return _sc_cumsum_call()(x)


# speedup vs baseline: 1.7055x; 1.7055x over previous
import functools

import jax
import jax.numpy as jnp
from jax import lax
from jax.experimental import pallas as pl
from jax.experimental.pallas import tpu as pltpu
from jax.experimental.pallas import tpu_sc as plsc

_N_ROWS, _N_COLS = 8192, 4096
_LANES = 16
_NUM_WORKERS = 32
_GROUPS = _N_ROWS // _NUM_WORKERS // _LANES
_CC = 2048


@functools.cache
def _sc_cumsum_call():
    mesh = plsc.VectorSubcoreMesh(core_axis_name="c", subcore_axis_name="s")

    @functools.partial(
        pl.kernel,
        mesh=mesh,
        compiler_params=pltpu.CompilerParams(
            needs_layout_passes=False, use_tc_tiling_on_sc=False),
        out_type=jax.ShapeDtypeStruct((_N_ROWS, _N_COLS), jnp.float32),
        scratch_types=[pltpu.VMEM((_LANES, _CC), jnp.float32),
                       pltpu.VMEM((_LANES, _CC), jnp.float32)],
    )
    def sc_cumsum(x_hbm, o_hbm, ibuf, obuf):
        wid = lax.axis_index("s") * 2 + lax.axis_index("c")

        def group_body(g, _):
            row0 = (wid * _GROUPS + g) * _LANES

            def chunk_body(c, carries):
                col0 = c * _CC
                pltpu.sync_copy(
                    x_hbm.at[pl.ds(row0, _LANES), pl.ds(col0, _CC)], ibuf)

                def col_body(j, carries):
                    new = []
                    for r in range(_LANES):
                        v = ibuf[r, pl.ds(j * _LANES, _LANES)]
                        s = plsc.cumsum(v) + carries[r]
                        obuf[r, pl.ds(j * _LANES, _LANES)] = s
                        new.append(s[_LANES - 1])
                    return tuple(new)

                carries = lax.fori_loop(
                    0, _CC // _LANES, col_body, carries)

                pltpu.sync_copy(
                    obuf, o_hbm.at[pl.ds(row0, _LANES), pl.ds(col0, _CC)])
                return carries

            lax.fori_loop(0, _N_COLS // _CC, chunk_body,
                          (jnp.float32(0),) * _LANES)
            return 0

        lax.fori_loop(0, _GROUPS, group_body, 0)

    return sc_cumsum


def kernel(x):
    return _sc_cumsum_call()(x)


# speedup vs baseline: 2.1568x; 1.2646x over previous
import functools

import jax
import jax.numpy as jnp
from jax import lax
from jax.experimental import pallas as pl
from jax.experimental.pallas import tpu as pltpu
from jax.experimental.pallas import tpu_sc as plsc

_N_ROWS, _N_COLS = 8192, 4096
_LANES = 16
_NUM_WORKERS = 32
_GROUPS = _N_ROWS // _NUM_WORKERS // _LANES
_CC = 2048


@functools.cache
def _sc_cumsum_call():
    mesh = plsc.VectorSubcoreMesh(core_axis_name="c", subcore_axis_name="s")

    @functools.partial(
        pl.kernel,
        mesh=mesh,
        compiler_params=pltpu.CompilerParams(needs_layout_passes=False),
        out_type=jax.ShapeDtypeStruct((_N_ROWS, _N_COLS), jnp.float32),
        scratch_types=[pltpu.VMEM((_LANES, _CC), jnp.float32),
                       pltpu.VMEM((_LANES, _CC), jnp.float32)],
    )
    def sc_cumsum(x_hbm, o_hbm, ibuf, obuf):
        wid = lax.axis_index("s") * 2 + lax.axis_index("c")

        def group_body(g, _):
            row0 = (wid * _GROUPS + g) * _LANES

            def chunk_body(c, carries):
                col0 = c * _CC
                pltpu.sync_copy(
                    x_hbm.at[pl.ds(row0, _LANES), pl.ds(col0, _CC)], ibuf)

                def col_body(j, carries):
                    new = []
                    for r in range(_LANES):
                        v = ibuf[r, pl.ds(j * _LANES, _LANES)]
                        s = plsc.cumsum(v) + carries[r]
                        obuf[r, pl.ds(j * _LANES, _LANES)] = s
                        new.append(s[_LANES - 1])
                    return tuple(new)

                carries = lax.fori_loop(
                    0, _CC // _LANES, col_body, carries)

                pltpu.sync_copy(
                    obuf, o_hbm.at[pl.ds(row0, _LANES), pl.ds(col0, _CC)])
                return carries

            lax.fori_loop(0, _N_COLS // _CC, chunk_body,
                          (jnp.float32(0),) * _LANES)
            return 0

        lax.fori_loop(0, _GROUPS, group_body, 0)

    return sc_cumsum


def kernel(x):
    return _sc_cumsum_call()(x)


# speedup vs baseline: 3.0160x; 1.3984x over previous
import functools

import jax
import jax.numpy as jnp
from jax import lax
from jax.experimental import pallas as pl
from jax.experimental.pallas import tpu as pltpu
from jax.experimental.pallas import tpu_sc as plsc

_N_ROWS, _N_COLS = 8192, 4096
_LANES = 16
_NUM_WORKERS = 32
_GROUPS = _N_ROWS // _NUM_WORKERS // _LANES
_CC = 1024
_CHUNKS = _N_COLS // _CC
_TILES = _GROUPS * _CHUNKS


@functools.cache
def _sc_cumsum_call():
    mesh = plsc.VectorSubcoreMesh(core_axis_name="c", subcore_axis_name="s")

    @functools.partial(
        pl.kernel,
        mesh=mesh,
        compiler_params=pltpu.CompilerParams(needs_layout_passes=False),
        out_type=jax.ShapeDtypeStruct((_N_ROWS, _N_COLS), jnp.float32),
        scratch_types=[pltpu.VMEM((_LANES, _CC), jnp.float32),
                       pltpu.VMEM((_LANES, _CC), jnp.float32),
                       pltpu.VMEM((_LANES, _CC), jnp.float32),
                       pltpu.VMEM((_LANES, _CC), jnp.float32),
                       pltpu.SemaphoreType.DMA,
                       pltpu.SemaphoreType.DMA,
                       pltpu.SemaphoreType.DMA,
                       pltpu.SemaphoreType.DMA],
    )
    def sc_cumsum(x_hbm, o_hbm, ibuf0, ibuf1, obuf0, obuf1,
                  sem_i0, sem_i1, sem_o0, sem_o1):
        wid = lax.axis_index("s") * 2 + lax.axis_index("c")
        ibufs, obufs = (ibuf0, ibuf1), (obuf0, obuf1)
        sems_i, sems_o = (sem_i0, sem_i1), (sem_o0, sem_o1)

        def tile_src(t):
            g = t // _CHUNKS
            c = lax.rem(t, _CHUNKS)
            row0 = (wid * _GROUPS + g) * _LANES
            return x_hbm.at[pl.ds(row0, _LANES), pl.ds(c * _CC, _CC)]

        def tile_dst(t):
            g = t // _CHUNKS
            c = lax.rem(t, _CHUNKS)
            row0 = (wid * _GROUPS + g) * _LANES
            return o_hbm.at[pl.ds(row0, _LANES), pl.ds(c * _CC, _CC)]

        pltpu.async_copy(tile_src(0), ibuf0, sem_i0)

        def step(i, carries):
            for b in range(2):
                t = 2 * i + b
                ib, ob = ibufs[b], obufs[b]

                @pl.when((t + 1 < _TILES) if b == 0 else (i < _TILES // 2 - 1))
                def _():
                    pltpu.async_copy(tile_src(t + 1), ibufs[1 - b],
                                     sems_i[1 - b])

                pltpu.make_async_copy(tile_src(t), ib, sems_i[b]).wait()

                @pl.when(i >= 1)
                def _():
                    pltpu.make_async_copy(ob, tile_dst(t), sems_o[b]).wait()

                maskf = (lax.rem(t, _CHUNKS) != 0).astype(jnp.float32)
                carries = tuple(cr * maskf for cr in carries)

                def col_body(j, carries):
                    new = []
                    for r in range(_LANES):
                        v = ib[r, pl.ds(j * _LANES, _LANES)]
                        s = plsc.cumsum(v) + carries[r]
                        ob[r, pl.ds(j * _LANES, _LANES)] = s
                        new.append(s[_LANES - 1])
                    return tuple(new)

                carries = lax.fori_loop(0, _CC // _LANES, col_body, carries)

                pltpu.async_copy(ob, tile_dst(t), sems_o[b])
            return carries

        lax.fori_loop(0, _TILES // 2, step, (jnp.float32(0),) * _LANES)

        pltpu.make_async_copy(obuf0, tile_dst(_TILES - 2), sem_o0).wait()
        pltpu.make_async_copy(obuf1, tile_dst(_TILES - 1), sem_o1).wait()

    return sc_cumsum


def kernel(x):
    return _sc_cumsum_call()(x)


# speedup vs baseline: 3.4756x; 1.1524x over previous
import functools

import jax
import jax.numpy as jnp
from jax import lax
from jax.experimental import pallas as pl
from jax.experimental.pallas import tpu as pltpu
from jax.experimental.pallas import tpu_sc as plsc

_N_ROWS, _N_COLS = 8192, 4096
_LANES = 16
_NUM_WORKERS = 32
_GROUPS = _N_ROWS // _NUM_WORKERS // _LANES
_CC = 1024
_CHUNKS = _N_COLS // _CC
_TILES = _GROUPS * _CHUNKS


@functools.cache
def _sc_cumsum_call():
    mesh = plsc.VectorSubcoreMesh(core_axis_name="c", subcore_axis_name="s")

    @functools.partial(
        pl.kernel,
        mesh=mesh,
        compiler_params=pltpu.CompilerParams(needs_layout_passes=False),
        out_type=jax.ShapeDtypeStruct((_N_ROWS, _N_COLS), jnp.float32),
        scratch_types=[pltpu.VMEM((_LANES, _CC), jnp.float32),
                       pltpu.VMEM((_LANES, _CC), jnp.float32),
                       pltpu.VMEM((_LANES, _CC), jnp.float32),
                       pltpu.VMEM((_LANES, _CC), jnp.float32),
                       pltpu.SemaphoreType.DMA,
                       pltpu.SemaphoreType.DMA,
                       pltpu.SemaphoreType.DMA,
                       pltpu.SemaphoreType.DMA],
    )
    def sc_cumsum(x_hbm, o_hbm, ibuf0, ibuf1, obuf0, obuf1,
                  sem_i0, sem_i1, sem_o0, sem_o1):
        wid = lax.axis_index("s") * 2 + lax.axis_index("c")
        ibufs, obufs = (ibuf0, ibuf1), (obuf0, obuf1)
        sems_i, sems_o = (sem_i0, sem_i1), (sem_o0, sem_o1)

        def tile_src(t):
            g = t // _CHUNKS
            c = lax.rem(t, _CHUNKS)
            row0 = (wid * _GROUPS + g) * _LANES
            return x_hbm.at[pl.ds(row0, _LANES), pl.ds(c * _CC, _CC)]

        def tile_dst(t):
            g = t // _CHUNKS
            c = lax.rem(t, _CHUNKS)
            row0 = (wid * _GROUPS + g) * _LANES
            return o_hbm.at[pl.ds(row0, _LANES), pl.ds(c * _CC, _CC)]

        pltpu.async_copy(tile_src(0), ibuf0, sem_i0)

        last = lax.iota(jnp.int32, _LANES) * 0 + (_LANES - 1)

        def step(i, carries):
            for b in range(2):
                t = 2 * i + b
                ib, ob = ibufs[b], obufs[b]

                @pl.when((t + 1 < _TILES) if b == 0 else (i < _TILES // 2 - 1))
                def _():
                    pltpu.async_copy(tile_src(t + 1), ibufs[1 - b],
                                     sems_i[1 - b])

                pltpu.make_async_copy(tile_src(t), ib, sems_i[b]).wait()

                @pl.when(i >= 1)
                def _():
                    pltpu.make_async_copy(ob, tile_dst(t), sems_o[b]).wait()

                maskf = (lax.rem(t, _CHUNKS) != 0).astype(jnp.float32)
                carries = tuple(cr * maskf for cr in carries)

                def col_body(j, carries):
                    new = []
                    for r in range(_LANES):
                        v = ib[r, pl.ds(j * _LANES, _LANES)]
                        s = plsc.cumsum(v) + carries[r]
                        ob[r, pl.ds(j * _LANES, _LANES)] = s
                        new.append(s.at[last].get(mode="promise_in_bounds"))
                    return tuple(new)

                carries = lax.fori_loop(0, _CC // _LANES, col_body, carries)

                pltpu.async_copy(ob, tile_dst(t), sems_o[b])
            return carries

        lax.fori_loop(0, _TILES // 2, step,
                      (jnp.zeros((_LANES,), jnp.float32),) * _LANES)

        pltpu.make_async_copy(obuf0, tile_dst(_TILES - 2), sem_o0).wait()
        pltpu.make_async_copy(obuf1, tile_dst(_TILES - 1), sem_o1).wait()

    return sc_cumsum


def kernel(x):
    return _sc_cumsum_call()(x)
